# Initial kernel scaffold; baseline (speedup 1.0000x reference)
import functools

import jax
import jax.numpy as jnp
from jax import lax
from jax.experimental import pallas as pl
from jax.experimental.pallas import tpu as pltpu
from jax.experimental.pallas import tpu_sc as plsc

N = 10000
E = 320000
D_IN = 128
D_E = 16
D = 256
NUM_LAYERS = 3
NUM_GRAPHS = 10

NC = 2
NS = 16
NSC = N // NC
PAD_ROWS = 8
ACC_ROWS = NSC + PAD_ROWS
CH = 80
EPT = E // NS
NCHUNK = EPT // CH
RPT = NSC // NS
TAIL = NSC - NS * RPT

_mesh = plsc.VectorSubcoreMesh(core_axis_name="c", subcore_axis_name="s")


def _acc_init_writeback(acc, hbm, base_node, s, to_acc):
    r0 = s * RPT
    if to_acc:
        pltpu.sync_copy(hbm.at[pl.ds(base_node + r0, RPT)], acc.at[pl.ds(r0, RPT)])
    else:
        pltpu.sync_copy(acc.at[pl.ds(r0, RPT)], hbm.at[pl.ds(base_node + r0, RPT)])

    @pl.when(s == NS - 1)
    def _():
        if to_acc:
            pltpu.sync_copy(hbm.at[pl.ds(base_node + NS * RPT, TAIL)],
                            acc.at[pl.ds(NS * RPT, TAIL)])
        else:
            pltpu.sync_copy(acc.at[pl.ds(NS * RPT, TAIL)],
                            hbm.at[pl.ds(base_node + NS * RPT, TAIL)])


def _local_dst(draw, dbuf, base_node):
    lanes = lax.iota(jnp.int32, 16)
    pad = NSC + (lanes & 7)
    for j in range(CH // 16):
        d = draw[pl.ds(j * 16, 16)]
        m = (d >= base_node) & (d < base_node + NSC)
        dbuf[pl.ds(j * 16, 16)] = jnp.where(m, d - base_node, pad)


def _node_agg_body(nf, init, src, dst, out, acc, sidx, draw, dbuf, rows, sem):
    c = lax.axis_index("c")
    s = lax.axis_index("s")
    base_node = c * NSC
    _acc_init_writeback(acc, init, base_node, s, to_acc=True)
    plsc.subcore_barrier()

    def chunk(i, carry):
        e0 = pl.multiple_of(s * EPT + i * CH, CH)
        pltpu.sync_copy(src.at[pl.ds(e0, CH)], sidx)
        pltpu.sync_copy(dst.at[pl.ds(e0, CH)], draw)
        _local_dst(draw, dbuf, base_node)
        pltpu.async_copy(nf.at[sidx], rows, sem).wait()
        pltpu.sync_copy(rows, acc.at[dbuf], add=True)
        return carry

    lax.fori_loop(0, NCHUNK, chunk, 0)
    plsc.subcore_barrier()
    _acc_init_writeback(acc, out, base_node, s, to_acc=False)


_node_agg = functools.partial(
    pl.kernel,
    out_type=jax.ShapeDtypeStruct((N, D), jnp.float32),
    mesh=_mesh,
    scratch_types=[
        pltpu.VMEM((CH,), jnp.int32),
        pltpu.VMEM((CH,), jnp.int32),
        pltpu.VMEM((CH,), jnp.int32),
        pltpu.VMEM((CH, D), jnp.float32),
        pltpu.VMEM_SHARED((ACC_ROWS, D), jnp.float32),
        pltpu.SemaphoreType.DMA,
    ],
)(_node_agg_body)


def _ea_agg_body(ea, dst, init, out, acc, draw, dbuf, rows, sem):
    c = lax.axis_index("c")
    s = lax.axis_index("s")
    base_node = c * NSC
    _acc_init_writeback(acc, init, base_node, s, to_acc=True)
    plsc.subcore_barrier()

    def chunk(i, carry):
        e0 = pl.multiple_of(s * EPT + i * CH, CH)
        pltpu.sync_copy(dst.at[pl.ds(e0, CH)], draw)
        _local_dst(draw, dbuf, base_node)
        pltpu.sync_copy(ea.at[pl.ds(e0, CH)], rows)
        pltpu.sync_copy(rows, acc.at[dbuf], add=True)
        return carry

    lax.fori_loop(0, NCHUNK, chunk, 0)
    plsc.subcore_barrier()
    _acc_init_writeback(acc, out, base_node, s, to_acc=False)


_ea_agg = functools.partial(
    pl.kernel,
    out_type=jax.ShapeDtypeStruct((N, 2 * D_E), jnp.float32),
    mesh=_mesh,
    scratch_types=[
        pltpu.VMEM((CH,), jnp.int32),
        pltpu.VMEM((CH,), jnp.int32),
        pltpu.VMEM((CH, 2 * D_E), jnp.float32),
        pltpu.VMEM_SHARED((ACC_ROWS, 2 * D_E), jnp.float32),
        pltpu.SemaphoreType.DMA,
    ],
)(_ea_agg_body)


_MB = 1000
_GRID = N // _MB


def _enc_body(x_ref, ea_ref, wn_ref, bn_ref, wa_ref, nf_ref, ef_ref):
    nf_ref[...] = (jnp.dot(x_ref[...], wn_ref[...],
                           preferred_element_type=jnp.float32)
                   + bn_ref[0:1, :])
    ef_ref[...] = jnp.dot(ea_ref[...], wa_ref[...],
                          preferred_element_type=jnp.float32)


def _encode(x, ea_agg, W_node, b_node8, W_aug):
    return pl.pallas_call(
        _enc_body,
        grid=(_GRID,),
        in_specs=[
            pl.BlockSpec((_MB, D_IN), lambda i: (i, 0)),
            pl.BlockSpec((_MB, 2 * D_E), lambda i: (i, 0)),
            pl.BlockSpec((D_IN, D), lambda i: (0, 0)),
            pl.BlockSpec((8, D), lambda i: (0, 0)),
            pl.BlockSpec((2 * D_E, D), lambda i: (0, 0)),
        ],
        out_specs=[
            pl.BlockSpec((_MB, D), lambda i: (i, 0)),
            pl.BlockSpec((_MB, D), lambda i: (i, 0)),
        ],
        out_shape=[
            jax.ShapeDtypeStruct((N, D), jnp.float32),
            jax.ShapeDtypeStruct((N, D), jnp.float32),
        ],
    )(x, ea_agg, W_node, b_node8, W_aug)


def _layer_body(ag_ref, nf_ref, w_ref, b_ref, o_ref):
    h = jnp.dot(ag_ref[...], w_ref[...], preferred_element_type=jnp.float32)
    o_ref[...] = jnp.maximum(h + b_ref[0:1, :], 0.0) + nf_ref[...]


def _layer(aggr, nf, W, b8):
    return pl.pallas_call(
        _layer_body,
        grid=(_GRID,),
        in_specs=[
            pl.BlockSpec((_MB, D), lambda i: (i, 0)),
            pl.BlockSpec((_MB, D), lambda i: (i, 0)),
            pl.BlockSpec((D, D), lambda i: (0, 0)),
            pl.BlockSpec((8, D), lambda i: (0, 0)),
        ],
        out_specs=pl.BlockSpec((_MB, D), lambda i: (i, 0)),
        out_shape=jax.ShapeDtypeStruct((N, D), jnp.float32),
    )(aggr, nf, W, b8)


def _last_body(ag_ref, nf_ref, w_ref, b_ref, wd_ref, bd_ref, o_ref):
    h = jnp.dot(ag_ref[...], w_ref[...], preferred_element_type=jnp.float32)
    h = jnp.maximum(h + b_ref[0:1, :], 0.0) + nf_ref[...]
    o_ref[...] = jnp.sum(h * wd_ref[0:1, :], axis=1, keepdims=True) + bd_ref[0, 0]


def _last(aggr, nf, W, b8, wd8, bd):
    return pl.pallas_call(
        _last_body,
        grid=(_GRID,),
        in_specs=[
            pl.BlockSpec((_MB, D), lambda i: (i, 0)),
            pl.BlockSpec((_MB, D), lambda i: (i, 0)),
            pl.BlockSpec((D, D), lambda i: (0, 0)),
            pl.BlockSpec((8, D), lambda i: (0, 0)),
            pl.BlockSpec((8, D), lambda i: (0, 0)),
            pl.BlockSpec(memory_space=pltpu.SMEM),
        ],
        out_specs=pl.BlockSpec((_MB, 1), lambda i: (i, 0)),
        out_shape=jax.ShapeDtypeStruct((N, 1), jnp.float32),
    )(aggr, nf, W, b8, wd8, bd)


def kernel(x, edge_index, edge_attr, W_node, b_node, W_edge, b_edge,
           W_layers, b_layers, W_dec, b_dec):
    src = edge_index[0]
    dst = edge_index[1]
    ea_pad = jnp.concatenate(
        [edge_attr,
         jnp.ones((E, 1), jnp.float32),
         jnp.zeros((E, D_E - 1), jnp.float32)], axis=1)
    W_aug = jnp.concatenate(
        [W_edge, b_edge[None, :], jnp.zeros((D_E - 1, D), jnp.float32)],
        axis=0)
    zeros_init = jnp.zeros((N, 2 * D_E), jnp.float32)
    b_node8 = jnp.broadcast_to(b_node[None, :], (8, D))
    wd8 = jnp.broadcast_to(W_dec.reshape(1, D), (8, D))
    bd = b_dec.reshape(1, 1)

    ea_agg = _ea_agg(ea_pad, dst, zeros_init)
    nf, ef_agg = _encode(x, ea_agg, W_node, b_node8, W_aug)
    out = None
    for l in range(NUM_LAYERS):
        aggr = _node_agg(nf, ef_agg, src, dst)
        b8 = jnp.broadcast_to(b_layers[l][None, :], (8, D))
        if l < NUM_LAYERS - 1:
            nf = _layer(aggr, nf, W_layers[l], b8)
        else:
            out = _last(aggr, nf, W_layers[l], b8, wd8, bd)
    return out.reshape(NUM_GRAPHS, N // NUM_GRAPHS, 1)


# speedup vs baseline: 1.3288x; 1.3288x over previous
import functools

import jax
import jax.numpy as jnp
from jax import lax
from jax.experimental import pallas as pl
from jax.experimental.pallas import tpu as pltpu
from jax.experimental.pallas import tpu_sc as plsc

N = 10000
E = 320000
D_IN = 128
D_E = 16
D = 256
NUM_LAYERS = 3
NUM_GRAPHS = 10

NC = 2
NS = 16
NT = NC * NS
LANES = 16

RPT = 312
RPT_LAST = N - (NT - 1) * RPT
ACC_ROWS = RPT_LAST + 8
PAD_ROW = RPT_LAST

SEG = 20000
NSEG = E // SEG
SCN = 10000
CAP = SEG + 16
TRASH = CAP
CH = 80

_mesh = plsc.VectorSubcoreMesh(core_axis_name="c", subcore_axis_name="s")
_SC_PARAMS = pltpu.CompilerParams(needs_layout_passes=False)


def _scan_body(src, dst, srcl, dstl, eidl, cnts,
               sbuf, dbuf, lsrc, ldst, leid, cbuf, sem):
    c = lax.axis_index("c")
    s = lax.axis_index("s")
    w = c * NS + s
    lanes = lax.iota(jnp.int32, LANES)
    lo = w * RPT
    hi = jnp.where(w == NT - 1, N, lo + RPT)
    pad_vec = jnp.full((LANES,), PAD_ROW, jnp.int32) + (lanes & 7)

    def prefill():
        def pf(i, carry):
            z16 = jnp.zeros((LANES,), jnp.int32)
            lsrc[pl.ds(i * LANES, LANES)] = z16
            ldst[pl.ds(i * LANES, LANES)] = pad_vec
            leid[pl.ds(i * LANES, LANES)] = z16
            return carry
        lax.fori_loop(0, CAP // LANES, pf, 0)

    prefill()
    for seg in range(NSEG):
        cnt = jnp.int32(0)
        for sub in range(SEG // SCN):
            e0 = seg * SEG + sub * SCN
            pltpu.sync_copy(src.at[pl.ds(e0, SCN)], sbuf)
            pltpu.sync_copy(dst.at[pl.ds(e0, SCN)], dbuf)

            def step(g, cnt, e0=e0):
                d = dbuf[pl.ds(g * LANES, LANES)]
                sv = sbuf[pl.ds(g * LANES, LANES)]
                m = (d >= lo) & (d < hi)
                mi = m.astype(jnp.int32)
                incl = plsc.cumsum(mi)
                pos = cnt + incl - mi
                tgt = jnp.where(m, pos, TRASH + lanes)
                eid = e0 + g * LANES + lanes
                plsc.store_scatter(lsrc, [tgt], sv)
                plsc.store_scatter(ldst, [tgt], d - lo)
                plsc.store_scatter(leid, [tgt], eid)
                return cnt + incl[LANES - 1]

            cnt = lax.fori_loop(0, SCN // LANES, step, cnt)
        cidx = jnp.full((LANES,), seg * LANES, jnp.int32) + lanes
        plsc.store_scatter(cbuf, [cidx], jnp.broadcast_to(cnt, (LANES,)))
        lbase = (w * NSEG + seg) * CAP
        pltpu.sync_copy(lsrc.at[pl.ds(0, CAP)], srcl.at[pl.ds(lbase, CAP)])
        pltpu.sync_copy(ldst.at[pl.ds(0, CAP)], dstl.at[pl.ds(lbase, CAP)])
        pltpu.sync_copy(leid.at[pl.ds(0, CAP)], eidl.at[pl.ds(lbase, CAP)])
        prefill()
    pltpu.sync_copy(cbuf, cnts.at[pl.ds(w * NSEG * LANES, NSEG * LANES)])


_scan = functools.partial(
    pl.kernel,
    compiler_params=_SC_PARAMS,
    out_type=[
        jax.ShapeDtypeStruct((NT * NSEG * CAP,), jnp.int32),
        jax.ShapeDtypeStruct((NT * NSEG * CAP,), jnp.int32),
        jax.ShapeDtypeStruct((NT * NSEG * CAP,), jnp.int32),
        jax.ShapeDtypeStruct((NT * NSEG * LANES,), jnp.int32),
    ],
    mesh=_mesh,
    scratch_types=[
        pltpu.VMEM((SCN,), jnp.int32),
        pltpu.VMEM((SCN,), jnp.int32),
        pltpu.VMEM((CAP + LANES,), jnp.int32),
        pltpu.VMEM((CAP + LANES,), jnp.int32),
        pltpu.VMEM((CAP + LANES,), jnp.int32),
        pltpu.VMEM((NSEG * LANES,), jnp.int32),
        pltpu.SemaphoreType.DMA,
    ],
)(_scan_body)


def _agg_body(table, gl, dstl, cnts, zsrc, out,
              acc, cv, sidx, didx, rows, sem):
    c = lax.axis_index("c")
    s = lax.axis_index("s")
    w = c * NS + s
    lo = w * RPT
    pltpu.sync_copy(zsrc.at[pl.ds(0, ACC_ROWS * D)], acc.at[pl.ds(0, ACC_ROWS * D)])
    pltpu.sync_copy(cnts.at[pl.ds(w * NSEG * LANES, NSEG * LANES)], cv)

    def segment(seg, carry):
        cnt = cv[pl.ds(seg * LANES, LANES)][0]
        nch = (cnt + (CH - 1)) // CH
        lbase = (w * NSEG + seg) * CAP

        def chunk(ch, carry1):
            pltpu.sync_copy(gl.at[pl.ds(lbase + ch * CH, CH)], sidx)
            pltpu.sync_copy(dstl.at[pl.ds(lbase + ch * CH, CH)], didx)
            pltpu.async_copy(table.at[sidx], rows, sem).wait()

            def grp(g, carry2):
                d16 = didx[pl.ds(g * LANES, LANES)]
                for lane in range(LANES):
                    base = d16[lane] * D
                    e = g * LANES + lane
                    for j in range(D // LANES):
                        plsc.addupdate(acc.at[pl.ds(base + j * LANES, LANES)],
                                       rows[e, pl.ds(j * LANES, LANES)])
                return carry2

            lax.fori_loop(0, CH // LANES, grp, 0)
            return carry1

        lax.fori_loop(0, nch, chunk, 0)
        return carry

    lax.fori_loop(0, NSEG, segment, 0)

    @pl.when(w < NT - 1)
    def _():
        pltpu.sync_copy(acc.at[pl.ds(0, RPT * D)], out.at[pl.ds(lo * D, RPT * D)])

    @pl.when(w == NT - 1)
    def _():
        pltpu.sync_copy(acc.at[pl.ds(0, RPT_LAST * D)],
                        out.at[pl.ds(lo * D, RPT_LAST * D)])


_agg = functools.partial(
    pl.kernel,
    compiler_params=_SC_PARAMS,
    out_type=jax.ShapeDtypeStruct((N * D,), jnp.float32),
    mesh=_mesh,
    scratch_types=[
        pltpu.VMEM((ACC_ROWS * D,), jnp.float32),
        pltpu.VMEM((NSEG * LANES,), jnp.int32),
        pltpu.VMEM((CH,), jnp.int32),
        pltpu.VMEM((CH,), jnp.int32),
        pltpu.VMEM((CH, D), jnp.float32),
        pltpu.SemaphoreType.DMA,
    ],
)(_agg_body)


_MB = 1000
_GRID = N // _MB


def _enc_body(x_ref, wn_ref, bn_ref, nf_ref):
    nf_ref[...] = (jnp.dot(x_ref[...], wn_ref[...],
                           preferred_element_type=jnp.float32)
                   + bn_ref[0:1, :])


def _encode(x, W_node, b_node8):
    return pl.pallas_call(
        _enc_body,
        grid=(_GRID,),
        in_specs=[
            pl.BlockSpec((_MB, D_IN), lambda i: (i, 0)),
            pl.BlockSpec((D_IN, D), lambda i: (0, 0)),
            pl.BlockSpec((8, D), lambda i: (0, 0)),
        ],
        out_specs=pl.BlockSpec((_MB, D), lambda i: (i, 0)),
        out_shape=jax.ShapeDtypeStruct((N, D), jnp.float32),
    )(x, W_node, b_node8)


_EB = 4000


def _ef_body(ea_ref, we_ref, be_ref, ef_ref):
    ef_ref[...] = (jnp.dot(ea_ref[...], we_ref[...],
                           preferred_element_type=jnp.float32)
                   + be_ref[0:1, :])


def _ef_mm(edge_attr, W_edge, b_edge8):
    return pl.pallas_call(
        _ef_body,
        grid=(E // _EB,),
        in_specs=[
            pl.BlockSpec((_EB, D_E), lambda i: (i, 0)),
            pl.BlockSpec((D_E, D), lambda i: (0, 0)),
            pl.BlockSpec((8, D), lambda i: (0, 0)),
        ],
        out_specs=pl.BlockSpec((_EB, D), lambda i: (i, 0)),
        out_shape=jax.ShapeDtypeStruct((E, D), jnp.float32),
    )(edge_attr, W_edge, b_edge8)


def _layer_body(ag_ref, ea_ref, nf_ref, w_ref, b_ref, out_ref):
    ag = ag_ref[...] + ea_ref[...]
    h = jnp.dot(ag, w_ref[...], preferred_element_type=jnp.float32)
    out_ref[...] = jnp.maximum(h + b_ref[0:1, :], 0.0) + nf_ref[...]


def _layer(ag, ea, nf, W, b8):
    return pl.pallas_call(
        _layer_body,
        grid=(_GRID,),
        in_specs=[
            pl.BlockSpec((_MB, D), lambda i: (i, 0)),
            pl.BlockSpec((_MB, D), lambda i: (i, 0)),
            pl.BlockSpec((_MB, D), lambda i: (i, 0)),
            pl.BlockSpec((D, D), lambda i: (0, 0)),
            pl.BlockSpec((8, D), lambda i: (0, 0)),
        ],
        out_specs=pl.BlockSpec((_MB, D), lambda i: (i, 0)),
        out_shape=jax.ShapeDtypeStruct((N, D), jnp.float32),
    )(ag, ea, nf, W, b8)


def _last_body(ag_ref, ea_ref, nf_ref, w_ref, b_ref, wd_ref, bd_ref, out_ref):
    ag = ag_ref[...] + ea_ref[...]
    h = jnp.dot(ag, w_ref[...], preferred_element_type=jnp.float32)
    h = jnp.maximum(h + b_ref[0:1, :], 0.0) + nf_ref[...]
    out_ref[...] = jnp.sum(h * wd_ref[0:1, :], axis=1, keepdims=True) + bd_ref[0, 0]


def _last(ag, ea, nf, W, b8, wd8, bd):
    return pl.pallas_call(
        _last_body,
        grid=(_GRID,),
        in_specs=[
            pl.BlockSpec((_MB, D), lambda i: (i, 0)),
            pl.BlockSpec((_MB, D), lambda i: (i, 0)),
            pl.BlockSpec((_MB, D), lambda i: (i, 0)),
            pl.BlockSpec((D, D), lambda i: (0, 0)),
            pl.BlockSpec((8, D), lambda i: (0, 0)),
            pl.BlockSpec((8, D), lambda i: (0, 0)),
            pl.BlockSpec(memory_space=pltpu.SMEM),
        ],
        out_specs=pl.BlockSpec((_MB, 1), lambda i: (i, 0)),
        out_shape=jax.ShapeDtypeStruct((N, 1), jnp.float32),
    )(ag, ea, nf, W, b8, wd8, bd)


def kernel(x, edge_index, edge_attr, W_node, b_node, W_edge, b_edge,
           W_layers, b_layers, W_dec, b_dec):
    src = edge_index[0]
    dst = edge_index[1]
    zsrc = jnp.zeros((ACC_ROWS * D,), jnp.float32)
    b_node8 = jnp.broadcast_to(b_node[None, :], (8, D))
    b_edge8 = jnp.broadcast_to(b_edge[None, :], (8, D))
    wd8 = jnp.broadcast_to(W_dec.reshape(1, D), (8, D))
    bd = b_dec.reshape(1, 1)

    srcl, dstl, eidl, cnts = _scan(src, dst)
    ef_full = _ef_mm(edge_attr, W_edge, b_edge8)
    ea_agg = _agg(ef_full, eidl, dstl, cnts, zsrc).reshape(N, D)
    nf = _encode(x, W_node, b_node8)
    out = None
    for l in range(NUM_LAYERS):
        aggr = _agg(nf, srcl, dstl, cnts, zsrc).reshape(N, D)
        b8 = jnp.broadcast_to(b_layers[l][None, :], (8, D))
        if l < NUM_LAYERS - 1:
            nf = _layer(aggr, ea_agg, nf, W_layers[l], b8)
        else:
            out = _last(aggr, ea_agg, nf, W_layers[l], b8, wd8, bd)
    return out.reshape(NUM_GRAPHS, N // NUM_GRAPHS, 1)
